# baseline (speedup 1.0000x reference)
import functools

import jax
import jax.numpy as jnp
from jax import lax
from jax.experimental import pallas as pl
from jax.experimental.pallas import tpu as pltpu
from jax.experimental.pallas import tpu_sc as plsc

D_MODEL = 1024
MAX_LEN = 8192
BATCH = 4
NUM_WORKERS = 32
ROWS_PER_WORKER = MAX_LEN // NUM_WORKERS
CHUNK_SIZES = (40, 40, 40, 40, 40, 40, 16)
CHUNK_OFFS = tuple(sum(CHUNK_SIZES[:i]) for i in range(len(CHUNK_SIZES)))
NUM_CHUNKS = len(CHUNK_SIZES)
BUF_ROWS = max(CHUNK_SIZES)
NBUF = 3


def _broadcast_body(table_hbm, out_hbm, *rest):
    bufs, (gsem, ssem) = rest[:NBUF], rest[NBUF:]
    cid = lax.axis_index("c")
    sid = lax.axis_index("s")
    wid = sid * 2 + cid
    base = wid * ROWS_PER_WORKER

    def gather(g):
        rows = CHUNK_SIZES[g]
        return pltpu.make_async_copy(
            table_hbm.at[pl.ds(base + CHUNK_OFFS[g], rows)],
            bufs[g % NBUF].at[pl.ds(0, rows)], gsem)

    gathers = [gather(g) for g in range(NUM_CHUNKS)]
    gathers[0].start()

    outstanding = []
    for g in range(NUM_CHUNKS):
        gathers[g].wait()
        scats = []
        for b in range(BATCH):
            sc = pltpu.make_async_copy(
                bufs[g % NBUF].at[pl.ds(0, CHUNK_SIZES[g])],
                out_hbm.at[b, pl.ds(base + CHUNK_OFFS[g], CHUNK_SIZES[g])],
                ssem)
            sc.start()
            scats.append(sc)
        outstanding.append(scats)
        if g + 1 < NUM_CHUNKS:
            if len(outstanding) >= NBUF:
                for sc in outstanding.pop(0):
                    sc.wait()
            gathers[g + 1].start()
    for scats in outstanding:
        for sc in scats:
            sc.wait()


@functools.partial(jax.jit, static_argnames=())
def _broadcast_table(table):
    mesh = plsc.VectorSubcoreMesh(core_axis_name="c", subcore_axis_name="s")
    run = pl.kernel(
        _broadcast_body,
        out_type=jax.ShapeDtypeStruct((BATCH, MAX_LEN, D_MODEL), jnp.float32),
        mesh=mesh,
        scratch_types=(
            [pltpu.VMEM((BUF_ROWS, D_MODEL), jnp.float32) for _ in range(NBUF)]
            + [pltpu.SemaphoreType.DMA, pltpu.SemaphoreType.DMA]
        ),
    )
    return run(table)


def kernel(x, table):
    del x
    return _broadcast_table(table)


# speedup vs baseline: 1.0176x; 1.0176x over previous
import functools

import jax
import jax.numpy as jnp
from jax import lax
from jax.experimental import pallas as pl
from jax.experimental.pallas import tpu as pltpu
from jax.experimental.pallas import tpu_sc as plsc

D_MODEL = 1024
MAX_LEN = 8192
BATCH = 4
NUM_WORKERS = 32
ROWS_PER_WORKER = MAX_LEN // NUM_WORKERS
CHUNK_SIZES = (32, 56, 56, 56, 56)
CHUNK_OFFS = tuple(sum(CHUNK_SIZES[:i]) for i in range(len(CHUNK_SIZES)))
NUM_CHUNKS = len(CHUNK_SIZES)
BUF_ROWS = max(CHUNK_SIZES)
NBUF = 2


def _broadcast_body(table_hbm, out_hbm, *rest):
    bufs, (gsem, ssem) = rest[:NBUF], rest[NBUF:]
    cid = lax.axis_index("c")
    sid = lax.axis_index("s")
    wid = sid * 2 + cid
    base = wid * ROWS_PER_WORKER

    def gather(g):
        rows = CHUNK_SIZES[g]
        return pltpu.make_async_copy(
            table_hbm.at[pl.ds(base + CHUNK_OFFS[g], rows)],
            bufs[g % NBUF].at[pl.ds(0, rows)], gsem)

    gathers = [gather(g) for g in range(NUM_CHUNKS)]
    gathers[0].start()

    outstanding = []
    for g in range(NUM_CHUNKS):
        gathers[g].wait()
        scats = []
        for b in range(BATCH):
            sc = pltpu.make_async_copy(
                bufs[g % NBUF].at[pl.ds(0, CHUNK_SIZES[g])],
                out_hbm.at[b, pl.ds(base + CHUNK_OFFS[g], CHUNK_SIZES[g])],
                ssem)
            sc.start()
            scats.append(sc)
        outstanding.append(scats)
        if g + 1 < NUM_CHUNKS:
            if len(outstanding) >= NBUF:
                for sc in outstanding.pop(0):
                    sc.wait()
            gathers[g + 1].start()
    for scats in outstanding:
        for sc in scats:
            sc.wait()


@functools.partial(jax.jit, static_argnames=())
def _broadcast_table(table):
    mesh = plsc.VectorSubcoreMesh(core_axis_name="c", subcore_axis_name="s")
    run = pl.kernel(
        _broadcast_body,
        out_type=jax.ShapeDtypeStruct((BATCH, MAX_LEN, D_MODEL), jnp.float32),
        mesh=mesh,
        scratch_types=(
            [pltpu.VMEM((BUF_ROWS, D_MODEL), jnp.float32) for _ in range(NBUF)]
            + [pltpu.SemaphoreType.DMA, pltpu.SemaphoreType.DMA]
        ),
    )
    return run(table)


def kernel(x, table):
    del x
    return _broadcast_table(table)
